# baseline (speedup 1.0000x reference)
import functools

import jax
import jax.numpy as jnp
from jax import lax
from jax.experimental import pallas as pl
from jax.experimental.pallas import tpu as pltpu
from jax.experimental.pallas import tpu_sc as plsc

L = 16
HW = 576
NV = HW // L
N = 32
C = 768
NW = 32
NCHUNK = 1
CC = C // NCHUNK
CPW = CC // NW

_INF = "INF"
_NINF = "NINF"


def _next_pow2(n):
    p = 1
    while p < n:
        p *= 2
    return p


def _vsort(v, desc):
    return plsc.sort_key_val(v, v, descending=desc)[0]


def _ce(a, b):
    if isinstance(a, str) or isinstance(b, str):
        if a is _INF:
            return b, a
        if b is _INF:
            return a, b
        if a is _NINF:
            return a, b
        return b, a
    return (jnp.minimum(a[0], b[0]), None), (jnp.maximum(a[0], b[0]), None)


def _bitonic_merge(S, desc):
    n = len(S)
    if n == 1:
        v = S[0]
        if isinstance(v, str):
            return [v]
        arr, sdir = v
        if sdir == desc:
            return [v]
        return [(_vsort(arr, desc), desc)]
    half = n // 2
    lo, hi = [], []
    for i in range(half):
        l, h = _ce(S[i], S[i + half])
        lo.append(l)
        hi.append(h)
    if desc:
        return _bitonic_merge(hi, desc) + _bitonic_merge(lo, desc)
    return _bitonic_merge(lo, desc) + _bitonic_merge(hi, desc)


def _merge(A, B, desc):
    p = _next_pow2(max(len(A), len(B)))
    if not desc:
        Apad = A + [_INF] * (p - len(A))
        Bpad = B + [_NINF] * (p - len(B))
    else:
        Apad = A + [_NINF] * (p - len(A))
        Bpad = B + [_INF] * (p - len(B))
    R = _bitonic_merge(Apad + Bpad, desc)
    out = [v for v in R if not isinstance(v, str)]
    assert len(out) == len(A) + len(B)
    return out


def _sort_run(vs, desc=False):
    if len(vs) == 1:
        return [(_vsort(vs[0][0], desc), desc)]
    h = len(vs) // 2
    A = _sort_run(vs[:h], desc)
    B = _sort_run(vs[h:], not desc)
    return _merge(A, B, desc)


def _lane_reduce_sum(v, lanes):
    for k in (1, 2, 4, 8):
        v = v + v.at[lanes ^ k].get(mode="promise_in_bounds")
    return v


def _sc_body(xt_hbm, w_hbm, out_hbm, xbuf, wbuf, obuf):
    wid = lax.axis_index("s") * 2 + lax.axis_index("c")
    lanes = lax.iota(jnp.int32, L)

    def chan_body(ci, _):
        c = wid * CPW + ci
        pltpu.sync_copy(w_hbm.at[c], wbuf)
        pltpu.sync_copy(xt_hbm.at[c], xbuf)

        s = jnp.zeros((L,), jnp.float32)
        for k in range(NV):
            wv = jnp.maximum(wbuf[pl.ds(k * L, L)], 0.0)
            wbuf[pl.ds(k * L, L)] = wv
            s = s + wv
        sinv = 1.0 / _lane_reduce_sum(s, lanes)

        def task_dot(n):
            base = n * HW
            vs = [(xbuf[pl.ds(base + k * L, L)], None) for k in range(NV)]
            srt = _sort_run(vs)
            dot = srt[0][0] * wbuf[pl.ds(0, L)]
            for k in range(1, NV):
                dot = dot + srt[k][0] * wbuf[pl.ds(k * L, L)]
            return _lane_reduce_sum(dot, lanes) * sinv

        def n_body(n, accs):
            acc0, acc1 = accs
            r0 = task_dot(n)
            r1 = task_dot(n + 8)
            r2 = task_dot(n + 16)
            r3 = task_dot(n + 24)
            acc0 = jnp.where(lanes == n, r0, acc0)
            acc0 = jnp.where(lanes == n + 8, r1, acc0)
            acc1 = jnp.where(lanes == n, r2, acc1)
            acc1 = jnp.where(lanes == n + 8, r3, acc1)
            return acc0, acc1

        z = jnp.zeros((L,), jnp.float32)
        acc0, acc1 = lax.fori_loop(0, N // 4, n_body, (z, z))
        obuf[pl.ds(0, L)] = acc0
        obuf[pl.ds(L, L)] = acc1
        pltpu.sync_copy(obuf, out_hbm.at[c])
        return ()

    lax.fori_loop(0, CPW, chan_body, ())


def _run(xt, wf):
    f = pl.kernel(
        _sc_body,
        out_type=jax.ShapeDtypeStruct((CC, N), jnp.float32),
        mesh=plsc.VectorSubcoreMesh(core_axis_name="c", subcore_axis_name="s"),
        compiler_params=pltpu.CompilerParams(needs_layout_passes=False),
        scratch_types=[
            pltpu.VMEM((N * HW,), jnp.float32),
            pltpu.VMEM((HW,), jnp.float32),
            pltpu.VMEM((N,), jnp.float32),
        ],
    )
    return f(xt, wf)


@jax.jit
def _pooled(x, w):
    xr = x.reshape(N, HW, C)
    wf = w[:, ::-1]
    outs = []
    for i in range(NCHUNK):
        xt = jnp.transpose(xr[:, :, i * CC:(i + 1) * CC], (2, 0, 1))
        outs.append(_run(xt.reshape(CC, N * HW), wf[i * CC:(i + 1) * CC]))
    out = jnp.concatenate(outs, axis=0)
    return out.T.reshape(N, 1, 1, C)


def kernel(x, ordinal_weights):
    return _pooled(x, ordinal_weights)


# speedup vs baseline: 2.1521x; 2.1521x over previous
import functools

import jax
import jax.numpy as jnp
from jax import lax
from jax.experimental import pallas as pl
from jax.experimental.pallas import tpu as pltpu
from jax.experimental.pallas import tpu_sc as plsc

L = 16
HW = 576
NV = HW // L
N = 32
C = 768
NW = 32
NCHUNK = 1
CC = C // NCHUNK
CPW = CC // NW

_INF = "INF"
_NINF = "NINF"


def _next_pow2(n):
    p = 1
    while p < n:
        p *= 2
    return p


def _vsort(v, desc):
    return tuple(plsc.sort_key_val(e, e, descending=desc)[0] for e in v)


def _ce(a, b):
    if isinstance(a, str) or isinstance(b, str):
        if a is _INF:
            return b, a
        if b is _INF:
            return a, b
        if a is _NINF:
            return a, b
        return b, a
    lo = tuple(jnp.minimum(x, y) for x, y in zip(a[0], b[0]))
    hi = tuple(jnp.maximum(x, y) for x, y in zip(a[0], b[0]))
    return (lo, None), (hi, None)


def _bitonic_merge(S, desc):
    n = len(S)
    if n == 1:
        v = S[0]
        if isinstance(v, str):
            return [v]
        arr, sdir = v
        if sdir == desc:
            return [v]
        return [(_vsort(arr, desc), desc)]
    half = n // 2
    lo, hi = [], []
    for i in range(half):
        l, h = _ce(S[i], S[i + half])
        lo.append(l)
        hi.append(h)
    if desc:
        return _bitonic_merge(hi, desc) + _bitonic_merge(lo, desc)
    return _bitonic_merge(lo, desc) + _bitonic_merge(hi, desc)


def _merge(A, B, desc):
    p = _next_pow2(max(len(A), len(B)))
    if not desc:
        Apad = A + [_INF] * (p - len(A))
        Bpad = B + [_NINF] * (p - len(B))
    else:
        Apad = A + [_NINF] * (p - len(A))
        Bpad = B + [_INF] * (p - len(B))
    R = _bitonic_merge(Apad + Bpad, desc)
    out = [v for v in R if not isinstance(v, str)]
    assert len(out) == len(A) + len(B)
    return out


def _sort_run(vs, desc=False):
    if len(vs) == 1:
        return [(_vsort(vs[0][0], desc), desc)]
    h = len(vs) // 2
    A = _sort_run(vs[:h], desc)
    B = _sort_run(vs[h:], not desc)
    return _merge(A, B, desc)


def _lane_reduce_sum(v, lanes):
    for k in (1, 2, 4, 8):
        v = v + v.at[lanes ^ k].get(mode="promise_in_bounds")
    return v


def _sc_body(xt_hbm, w_hbm, out_hbm, xbuf, wbuf, obuf):
    wid = lax.axis_index("s") * 2 + lax.axis_index("c")
    lanes = lax.iota(jnp.int32, L)

    def chan_body(ci, _):
        c = wid * CPW + ci
        pltpu.sync_copy(w_hbm.at[c], wbuf)
        pltpu.sync_copy(xt_hbm.at[c], xbuf)

        s = jnp.zeros((L,), jnp.float32)
        for k in range(NV):
            wv = jnp.maximum(wbuf[pl.ds(k * L, L)], 0.0)
            wbuf[pl.ds(k * L, L)] = wv
            s = s + wv
        sinv = 1.0 / _lane_reduce_sum(s, lanes)

        def n_body(n, accs):
            acc0, acc1 = accs
            ba, bb = n * HW, (n + 16) * HW
            vs = [
                ((xbuf[pl.ds(ba + k * L, L)], xbuf[pl.ds(bb + k * L, L)]),
                 None)
                for k in range(NV)
            ]
            srt = _sort_run(vs)
            wv = wbuf[pl.ds(0, L)]
            da, db = srt[0][0][0] * wv, srt[0][0][1] * wv
            for k in range(1, NV):
                wv = wbuf[pl.ds(k * L, L)]
                da = da + srt[k][0][0] * wv
                db = db + srt[k][0][1] * wv
            ra = _lane_reduce_sum(da, lanes) * sinv
            rb = _lane_reduce_sum(db, lanes) * sinv
            acc0 = jnp.where(lanes == n, ra, acc0)
            acc1 = jnp.where(lanes == n, rb, acc1)
            return acc0, acc1

        z = jnp.zeros((L,), jnp.float32)
        acc0, acc1 = lax.fori_loop(0, N // 2, n_body, (z, z))
        obuf[pl.ds(0, L)] = acc0
        obuf[pl.ds(L, L)] = acc1
        pltpu.sync_copy(obuf, out_hbm.at[c])
        return ()

    lax.fori_loop(0, CPW, chan_body, ())


def _run(xt, wf):
    f = pl.kernel(
        _sc_body,
        out_type=jax.ShapeDtypeStruct((CC, N), jnp.float32),
        mesh=plsc.VectorSubcoreMesh(core_axis_name="c", subcore_axis_name="s"),
        compiler_params=pltpu.CompilerParams(needs_layout_passes=False),
        scratch_types=[
            pltpu.VMEM((N * HW,), jnp.float32),
            pltpu.VMEM((HW,), jnp.float32),
            pltpu.VMEM((N,), jnp.float32),
        ],
    )
    return f(xt, wf)


@jax.jit
def _pooled(x, w):
    xr = x.reshape(N, HW, C)
    wf = w[:, ::-1]
    outs = []
    for i in range(NCHUNK):
        xt = jnp.transpose(xr[:, :, i * CC:(i + 1) * CC], (2, 0, 1))
        outs.append(_run(xt.reshape(CC, N * HW), wf[i * CC:(i + 1) * CC]))
    out = jnp.concatenate(outs, axis=0)
    return out.T.reshape(N, 1, 1, C)


def kernel(x, ordinal_weights):
    return _pooled(x, ordinal_weights)


# speedup vs baseline: 2.2868x; 1.0626x over previous
import functools

import jax
import jax.numpy as jnp
from jax import lax
from jax.experimental import pallas as pl
from jax.experimental.pallas import tpu as pltpu
from jax.experimental.pallas import tpu_sc as plsc

L = 16
HW = 576
NV = HW // L
N = 32
C = 768
NW = 32
NCHUNK = 1
CC = C // NCHUNK
CPW = CC // NW

_INF = "INF"
_NINF = "NINF"


def _next_pow2(n):
    p = 1
    while p < n:
        p *= 2
    return p


def _vsort(v, desc):
    return tuple(plsc.sort_key_val(e, e, descending=desc)[0] for e in v)


def _ce(a, b):
    if isinstance(a, str) or isinstance(b, str):
        if a is _INF:
            return b, a
        if b is _INF:
            return a, b
        if a is _NINF:
            return a, b
        return b, a
    lo = tuple(jnp.minimum(x, y) for x, y in zip(a[0], b[0]))
    hi = tuple(jnp.maximum(x, y) for x, y in zip(a[0], b[0]))
    return (lo, None), (hi, None)


def _bitonic_merge(S, desc):
    n = len(S)
    if n == 1:
        v = S[0]
        if isinstance(v, str):
            return [v]
        arr, sdir = v
        if sdir == desc:
            return [v]
        return [(_vsort(arr, desc), desc)]
    half = n // 2
    lo, hi = [], []
    for i in range(half):
        l, h = _ce(S[i], S[i + half])
        lo.append(l)
        hi.append(h)
    if desc:
        return _bitonic_merge(hi, desc) + _bitonic_merge(lo, desc)
    return _bitonic_merge(lo, desc) + _bitonic_merge(hi, desc)


def _merge(A, B, desc):
    p = _next_pow2(max(len(A), len(B)))
    if not desc:
        Apad = A + [_INF] * (p - len(A))
        Bpad = B + [_NINF] * (p - len(B))
    else:
        Apad = A + [_NINF] * (p - len(A))
        Bpad = B + [_INF] * (p - len(B))
    R = _bitonic_merge(Apad + Bpad, desc)
    out = [v for v in R if not isinstance(v, str)]
    assert len(out) == len(A) + len(B)
    return out


def _sort_run(vs, desc=False):
    if len(vs) == 1:
        return [(_vsort(vs[0][0], desc), desc)]
    h = len(vs) // 2
    A = _sort_run(vs[:h], desc)
    B = _sort_run(vs[h:], not desc)
    return _merge(A, B, desc)


def _lane_reduce_sum(v, lanes):
    for k in (1, 2, 4, 8):
        v = v + v.at[lanes ^ k].get(mode="promise_in_bounds")
    return v


def _sc_body(xt_hbm, w_hbm, out_hbm, xbuf0, xbuf1, wbuf, obuf, sem0, sem1):
    wid = lax.axis_index("s") * 2 + lax.axis_index("c")
    lanes = lax.iota(jnp.int32, L)
    c0 = wid * CPW

    def process(c, xbuf):
        pltpu.sync_copy(w_hbm.at[c], wbuf)

        s = jnp.zeros((L,), jnp.float32)
        for k in range(NV):
            wv = jnp.maximum(wbuf[pl.ds(k * L, L)], 0.0)
            wbuf[pl.ds(k * L, L)] = wv
            s = s + wv
        sinv = 1.0 / _lane_reduce_sum(s, lanes)

        def n_body(n, accs):
            acc0, acc1 = accs
            ba, bb = n * HW, (n + 16) * HW
            vs = [
                ((xbuf[pl.ds(ba + k * L, L)], xbuf[pl.ds(bb + k * L, L)]),
                 None)
                for k in range(NV)
            ]
            srt = _sort_run(vs)
            wv = wbuf[pl.ds(0, L)]
            da, db = srt[0][0][0] * wv, srt[0][0][1] * wv
            for k in range(1, NV):
                wv = wbuf[pl.ds(k * L, L)]
                da = da + srt[k][0][0] * wv
                db = db + srt[k][0][1] * wv
            ra = _lane_reduce_sum(da, lanes) * sinv
            rb = _lane_reduce_sum(db, lanes) * sinv
            acc0 = jnp.where(lanes == n, ra, acc0)
            acc1 = jnp.where(lanes == n, rb, acc1)
            return acc0, acc1

        z = jnp.zeros((L,), jnp.float32)
        acc0, acc1 = lax.fori_loop(0, N // 2, n_body, (z, z))
        obuf[pl.ds(0, L)] = acc0
        obuf[pl.ds(L, L)] = acc1
        pltpu.sync_copy(obuf, out_hbm.at[c])

    pltpu.async_copy(xt_hbm.at[c0], xbuf0, sem0)

    def chan_pair(i, _):
        ca = c0 + 2 * i
        pltpu.async_copy(xt_hbm.at[ca + 1], xbuf1, sem1)
        pltpu.make_async_copy(xt_hbm.at[ca], xbuf0, sem0).wait()
        process(ca, xbuf0)

        @pl.when(i < CPW // 2 - 1)
        def _():
            pltpu.async_copy(xt_hbm.at[ca + 2], xbuf0, sem0)

        pltpu.make_async_copy(xt_hbm.at[ca + 1], xbuf1, sem1).wait()
        process(ca + 1, xbuf1)
        return ()

    lax.fori_loop(0, CPW // 2, chan_pair, ())


def _run(xt, wf):
    f = pl.kernel(
        _sc_body,
        out_type=jax.ShapeDtypeStruct((CC, N), jnp.float32),
        mesh=plsc.VectorSubcoreMesh(core_axis_name="c", subcore_axis_name="s"),
        compiler_params=pltpu.CompilerParams(needs_layout_passes=False),
        scratch_types=[
            pltpu.VMEM((N * HW,), jnp.float32),
            pltpu.VMEM((N * HW,), jnp.float32),
            pltpu.VMEM((HW,), jnp.float32),
            pltpu.VMEM((N,), jnp.float32),
            pltpu.SemaphoreType.DMA,
            pltpu.SemaphoreType.DMA,
        ],
    )
    return f(xt, wf)


@jax.jit
def _pooled(x, w):
    xr = x.reshape(N, HW, C)
    wf = w[:, ::-1]
    outs = []
    for i in range(NCHUNK):
        xt = jnp.transpose(xr[:, :, i * CC:(i + 1) * CC], (2, 0, 1))
        outs.append(_run(xt.reshape(CC, N * HW), wf[i * CC:(i + 1) * CC]))
    out = jnp.concatenate(outs, axis=0)
    return out.T.reshape(N, 1, 1, C)


def kernel(x, ordinal_weights):
    return _pooled(x, ordinal_weights)


# speedup vs baseline: 2.2875x; 1.0003x over previous
import jax
import jax.numpy as jnp
from jax import lax
from jax.experimental import pallas as pl
from jax.experimental.pallas import tpu as pltpu
from jax.experimental.pallas import tpu_sc as plsc

L = 16
HW = 576
NV = HW // L
N = 32
C = 768
NW = 32
CPW = C // NW

_INF = "INF"
_NINF = "NINF"


def _next_pow2(n):
    p = 1
    while p < n:
        p *= 2
    return p


def _vsort(v, desc):
    return tuple(plsc.sort_key_val(e, e, descending=desc)[0] for e in v)


def _ce(a, b):
    if isinstance(a, str) or isinstance(b, str):
        if a is _INF:
            return b, a
        if b is _INF:
            return a, b
        if a is _NINF:
            return a, b
        return b, a
    lo = tuple(jnp.minimum(x, y) for x, y in zip(a[0], b[0]))
    hi = tuple(jnp.maximum(x, y) for x, y in zip(a[0], b[0]))
    return (lo, None), (hi, None)


def _bitonic_merge(S, desc):
    n = len(S)
    if n == 1:
        v = S[0]
        if isinstance(v, str):
            return [v]
        arr, sdir = v
        if sdir == desc:
            return [v]
        return [(_vsort(arr, desc), desc)]
    half = n // 2
    lo, hi = [], []
    for i in range(half):
        l, h = _ce(S[i], S[i + half])
        lo.append(l)
        hi.append(h)
    if desc:
        return _bitonic_merge(hi, desc) + _bitonic_merge(lo, desc)
    return _bitonic_merge(lo, desc) + _bitonic_merge(hi, desc)


def _merge(A, B, desc):
    p = _next_pow2(max(len(A), len(B)))
    if not desc:
        Apad = A + [_INF] * (p - len(A))
        Bpad = B + [_NINF] * (p - len(B))
    else:
        Apad = A + [_NINF] * (p - len(A))
        Bpad = B + [_INF] * (p - len(B))
    R = _bitonic_merge(Apad + Bpad, desc)
    out = [v for v in R if not isinstance(v, str)]
    assert len(out) == len(A) + len(B)
    return out


def _sort_run(vs, desc=False):
    if len(vs) == 1:
        return [(_vsort(vs[0][0], desc), desc)]
    h = len(vs) // 2
    A = _sort_run(vs[:h], desc)
    B = _sort_run(vs[h:], not desc)
    return _merge(A, B, desc)


def _lane_reduce_sum(v, lanes):
    for k in (1, 2, 4, 8):
        v = v + v.at[lanes ^ k].get(mode="promise_in_bounds")
    return v


def _sc_body(xt_hbm, w_hbm, out_hbm, xbuf0, xbuf1, wbuf, obuf, sem0, sem1):
    wid = lax.axis_index("s") * 2 + lax.axis_index("c")
    lanes = lax.iota(jnp.int32, L)
    c0 = wid * CPW

    def process(c, xbuf):
        pltpu.sync_copy(w_hbm.at[c], wbuf)

        s = jnp.zeros((L,), jnp.float32)
        for k in range(NV):
            wv = jnp.maximum(wbuf[pl.ds(k * L, L)], 0.0)
            wbuf[pl.ds(k * L, L)] = wv
            s = s + wv
        sinv = 1.0 / _lane_reduce_sum(s, lanes)

        def n_body(n, accs):
            acc0, acc1 = accs
            ba, bb = n * HW, (n + 16) * HW
            vs = [
                ((xbuf[pl.ds(ba + k * L, L)], xbuf[pl.ds(bb + k * L, L)]),
                 None)
                for k in range(NV)
            ]
            srt = _sort_run(vs)
            wv = wbuf[pl.ds(0, L)]
            da, db = srt[0][0][0] * wv, srt[0][0][1] * wv
            for k in range(1, NV):
                wv = wbuf[pl.ds(k * L, L)]
                da = da + srt[k][0][0] * wv
                db = db + srt[k][0][1] * wv
            ra = _lane_reduce_sum(da, lanes) * sinv
            rb = _lane_reduce_sum(db, lanes) * sinv
            acc0 = jnp.where(lanes == n, ra, acc0)
            acc1 = jnp.where(lanes == n, rb, acc1)
            return acc0, acc1

        z = jnp.zeros((L,), jnp.float32)
        acc0, acc1 = lax.fori_loop(0, N // 2, n_body, (z, z))
        obuf[pl.ds(0, L)] = acc0
        obuf[pl.ds(L, L)] = acc1
        pltpu.sync_copy(obuf, out_hbm.at[c])

    pltpu.async_copy(xt_hbm.at[c0], xbuf0, sem0)

    def chan_pair(i, _):
        ca = c0 + 2 * i
        pltpu.async_copy(xt_hbm.at[ca + 1], xbuf1, sem1)
        pltpu.make_async_copy(xt_hbm.at[ca], xbuf0, sem0).wait()
        process(ca, xbuf0)

        @pl.when(i < CPW // 2 - 1)
        def _():
            pltpu.async_copy(xt_hbm.at[ca + 2], xbuf0, sem0)

        pltpu.make_async_copy(xt_hbm.at[ca + 1], xbuf1, sem1).wait()
        process(ca + 1, xbuf1)
        return ()

    lax.fori_loop(0, CPW // 2, chan_pair, ())


def _run(xt, wf):
    f = pl.kernel(
        _sc_body,
        out_type=jax.ShapeDtypeStruct((C, N), jnp.float32),
        mesh=plsc.VectorSubcoreMesh(core_axis_name="c", subcore_axis_name="s"),
        compiler_params=pltpu.CompilerParams(needs_layout_passes=False),
        scratch_types=[
            pltpu.VMEM((N * HW,), jnp.float32),
            pltpu.VMEM((N * HW,), jnp.float32),
            pltpu.VMEM((HW,), jnp.float32),
            pltpu.VMEM((N,), jnp.float32),
            pltpu.SemaphoreType.DMA,
            pltpu.SemaphoreType.DMA,
        ],
    )
    return f(xt, wf)


@jax.jit
def _pooled(x, w):
    xt = jnp.transpose(x.reshape(N, HW, C), (2, 0, 1)).reshape(C, N * HW)
    out = _run(xt, w[:, ::-1])
    return out.T.reshape(N, 1, 1, C)


def kernel(x, ordinal_weights):
    return _pooled(x, ordinal_weights)
